# baseline (speedup 1.0000x reference)
import jax
import jax.numpy as jnp
from jax.experimental import pallas as pl
from jax.experimental.pallas import tpu as pltpu

_ROWS = 8 * 3136
_COLS = 768
_CH = 1280
_NCH = -(-_ROWS // _CH)
_CHUNKS = [(i * _CH, min(_CH, _ROWS - i * _CH)) for i in range(_NCH)]
_NBUF = 16
_NSPLIT = 1
_LOOK = 4


def _ring_body(x_hbm, o_hbm, *refs):
    bufs = refs[:_NBUF]
    ise = refs[_NBUF:2 * _NBUF]
    ose = refs[2 * _NBUF:]

    def _splits(n):
        q = (n // _NSPLIT) // 8 * 8
        cuts = [j * q for j in range(_NSPLIT)] + [n]
        return [(cuts[j], cuts[j + 1] - cuts[j]) for j in range(_NSPLIT)]

    def cin(i):
        off, n = _CHUNKS[i]
        b = i % _NBUF
        return [
            pltpu.make_async_copy(
                x_hbm.at[pl.ds(off + s, m)], bufs[b].at[pl.ds(s, m)], ise[b])
            for s, m in _splits(n)]

    def cout(i):
        off, n = _CHUNKS[i]
        b = i % _NBUF
        return [
            pltpu.make_async_copy(
                bufs[b].at[pl.ds(s, m)], o_hbm.at[pl.ds(off + s, m)], ose[b])
            for s, m in _splits(n)]

    def start(cps):
        for cp in cps:
            cp.start()

    def wait(cps):
        for cp in cps:
            cp.wait()

    for i in range(_LOOK):
        start(cin(i))
    for i in range(_NCH):
        wait(cin(i))
        start(cout(i))
        j = i + _LOOK
        if j < _NCH:
            if j >= _NBUF:
                wait(cout(j - _NBUF))
            start(cin(j))
    for i in range(_NCH - _NBUF, _NCH):
        wait(cout(i))


def kernel(x):
    flat = x.reshape(_ROWS, _COLS)
    out = pl.pallas_call(
        _ring_body,
        out_shape=jax.ShapeDtypeStruct(flat.shape, flat.dtype),
        in_specs=[pl.BlockSpec(memory_space=pl.ANY)],
        out_specs=pl.BlockSpec(memory_space=pl.ANY),
        scratch_shapes=(
            [pltpu.VMEM((_CH, _COLS), jnp.float32) for _ in range(_NBUF)]
            + [pltpu.SemaphoreType.DMA] * (2 * _NBUF)
        ),
        compiler_params=pltpu.CompilerParams(vmem_limit_bytes=128 * 1024 * 1024),
    )(flat)
    return out.reshape(x.shape)


# speedup vs baseline: 1.0021x; 1.0021x over previous
import jax
import jax.numpy as jnp
from jax.experimental import pallas as pl
from jax.experimental.pallas import tpu as pltpu

_ROWS = 8 * 3136
_COLS = 768
_CH = 1280
_NCH = -(-_ROWS // _CH)
_CHUNKS = [(i * _CH, min(_CH, _ROWS - i * _CH)) for i in range(_NCH)]
_NBUF = 16
_NSPLIT = 1
_LOOK = 8


def _ring_body(x_hbm, o_hbm, *refs):
    bufs = refs[:_NBUF]
    ise = refs[_NBUF:2 * _NBUF]
    ose = refs[2 * _NBUF:]

    def _splits(n):
        q = (n // _NSPLIT) // 8 * 8
        cuts = [j * q for j in range(_NSPLIT)] + [n]
        return [(cuts[j], cuts[j + 1] - cuts[j]) for j in range(_NSPLIT)]

    def cin(i):
        off, n = _CHUNKS[i]
        b = i % _NBUF
        return [
            pltpu.make_async_copy(
                x_hbm.at[pl.ds(off + s, m)], bufs[b].at[pl.ds(s, m)], ise[b])
            for s, m in _splits(n)]

    def cout(i):
        off, n = _CHUNKS[i]
        b = i % _NBUF
        return [
            pltpu.make_async_copy(
                bufs[b].at[pl.ds(s, m)], o_hbm.at[pl.ds(off + s, m)], ose[b])
            for s, m in _splits(n)]

    def start(cps):
        for cp in cps:
            cp.start()

    def wait(cps):
        for cp in cps:
            cp.wait()

    for i in range(_LOOK):
        start(cin(i))
    for i in range(_NCH):
        wait(cin(i))
        start(cout(i))
        j = i + _LOOK
        if j < _NCH:
            if j >= _NBUF:
                wait(cout(j - _NBUF))
            start(cin(j))
    for i in range(_NCH - _NBUF, _NCH):
        wait(cout(i))


def kernel(x):
    flat = x.reshape(_ROWS, _COLS)
    out = pl.pallas_call(
        _ring_body,
        out_shape=jax.ShapeDtypeStruct(flat.shape, flat.dtype),
        in_specs=[pl.BlockSpec(memory_space=pl.ANY)],
        out_specs=pl.BlockSpec(memory_space=pl.ANY),
        scratch_shapes=(
            [pltpu.VMEM((_CH, _COLS), jnp.float32) for _ in range(_NBUF)]
            + [pltpu.SemaphoreType.DMA] * (2 * _NBUF)
        ),
        compiler_params=pltpu.CompilerParams(vmem_limit_bytes=128 * 1024 * 1024),
    )(flat)
    return out.reshape(x.shape)


# speedup vs baseline: 1.0046x; 1.0025x over previous
import jax
import jax.numpy as jnp
from jax.experimental import pallas as pl
from jax.experimental.pallas import tpu as pltpu

_ROWS = 8 * 3136
_COLS = 768
_CH = 1280
_NCH = -(-_ROWS // _CH)
_CHUNKS = [(i * _CH, min(_CH, _ROWS - i * _CH)) for i in range(_NCH)]
_NBUF = 16
_NSPLIT = 1
_LOOK = 15


def _ring_body(x_hbm, o_hbm, *refs):
    bufs = refs[:_NBUF]
    ise = refs[_NBUF:2 * _NBUF]
    ose = refs[2 * _NBUF:]

    def _splits(n):
        q = (n // _NSPLIT) // 8 * 8
        cuts = [j * q for j in range(_NSPLIT)] + [n]
        return [(cuts[j], cuts[j + 1] - cuts[j]) for j in range(_NSPLIT)]

    def cin(i):
        off, n = _CHUNKS[i]
        b = i % _NBUF
        return [
            pltpu.make_async_copy(
                x_hbm.at[pl.ds(off + s, m)], bufs[b].at[pl.ds(s, m)], ise[b])
            for s, m in _splits(n)]

    def cout(i):
        off, n = _CHUNKS[i]
        b = i % _NBUF
        return [
            pltpu.make_async_copy(
                bufs[b].at[pl.ds(s, m)], o_hbm.at[pl.ds(off + s, m)], ose[b])
            for s, m in _splits(n)]

    def start(cps):
        for cp in cps:
            cp.start()

    def wait(cps):
        for cp in cps:
            cp.wait()

    for i in range(_LOOK):
        start(cin(i))
    for i in range(_NCH):
        wait(cin(i))
        start(cout(i))
        j = i + _LOOK
        if j < _NCH:
            if j >= _NBUF:
                wait(cout(j - _NBUF))
            start(cin(j))
    for i in range(_NCH - _NBUF, _NCH):
        wait(cout(i))


def kernel(x):
    flat = x.reshape(_ROWS, _COLS)
    out = pl.pallas_call(
        _ring_body,
        out_shape=jax.ShapeDtypeStruct(flat.shape, flat.dtype),
        in_specs=[pl.BlockSpec(memory_space=pl.ANY)],
        out_specs=pl.BlockSpec(memory_space=pl.ANY),
        scratch_shapes=(
            [pltpu.VMEM((_CH, _COLS), jnp.float32) for _ in range(_NBUF)]
            + [pltpu.SemaphoreType.DMA] * (2 * _NBUF)
        ),
        compiler_params=pltpu.CompilerParams(vmem_limit_bytes=128 * 1024 * 1024),
    )(flat)
    return out.reshape(x.shape)


# speedup vs baseline: 1.0564x; 1.0515x over previous
import jax
import jax.numpy as jnp
from jax.experimental import pallas as pl
from jax.experimental.pallas import tpu as pltpu

_ROWS = 8 * 3136
_COLS = 768
_CH = 2560
_NCH = -(-_ROWS // _CH)
_CHUNKS = [(i * _CH, min(_CH, _ROWS - i * _CH)) for i in range(_NCH)]
_NBUF = 8
_NSPLIT = 4
_LOOK = 7


def _ring_body(x_hbm, o_hbm, *refs):
    bufs = refs[:_NBUF]
    ise = refs[_NBUF:2 * _NBUF]
    ose = refs[2 * _NBUF:]

    def _splits(n):
        q = (n // _NSPLIT) // 8 * 8
        cuts = [j * q for j in range(_NSPLIT)] + [n]
        return [(cuts[j], cuts[j + 1] - cuts[j]) for j in range(_NSPLIT)]

    def cin(i):
        off, n = _CHUNKS[i]
        b = i % _NBUF
        return [
            pltpu.make_async_copy(
                x_hbm.at[pl.ds(off + s, m)], bufs[b].at[pl.ds(s, m)], ise[b])
            for s, m in _splits(n)]

    def cout(i):
        off, n = _CHUNKS[i]
        b = i % _NBUF
        return [
            pltpu.make_async_copy(
                bufs[b].at[pl.ds(s, m)], o_hbm.at[pl.ds(off + s, m)], ose[b])
            for s, m in _splits(n)]

    def start(cps):
        for cp in cps:
            cp.start()

    def wait(cps):
        for cp in cps:
            cp.wait()

    for i in range(_LOOK):
        start(cin(i))
    for i in range(_NCH):
        wait(cin(i))
        start(cout(i))
        j = i + _LOOK
        if j < _NCH:
            if j >= _NBUF:
                wait(cout(j - _NBUF))
            start(cin(j))
    for i in range(_NCH - _NBUF, _NCH):
        wait(cout(i))


def kernel(x):
    flat = x.reshape(_ROWS, _COLS)
    out = pl.pallas_call(
        _ring_body,
        out_shape=jax.ShapeDtypeStruct(flat.shape, flat.dtype),
        in_specs=[pl.BlockSpec(memory_space=pl.ANY)],
        out_specs=pl.BlockSpec(memory_space=pl.ANY),
        scratch_shapes=(
            [pltpu.VMEM((_CH, _COLS), jnp.float32) for _ in range(_NBUF)]
            + [pltpu.SemaphoreType.DMA] * (2 * _NBUF)
        ),
        compiler_params=pltpu.CompilerParams(vmem_limit_bytes=128 * 1024 * 1024),
    )(flat)
    return out.reshape(x.shape)
